# baseline (speedup 1.0000x reference)
import functools

import jax
import jax.numpy as jnp
from jax.experimental import pallas as pl
from jax.experimental.pallas import tpu as pltpu

_T = 128

_NEG = -jnp.inf
_CT = jnp.bfloat16


def _ctx_kernel(x_ref, y_ref, w_ref, b_ref, o_ref, ctx_sref, *, T, L, H):
    tp = pl.program_id(1)
    tq = pl.program_id(2)
    p0 = tp * T
    q0 = tq * T

    xrow = x_ref[0]
    xt = x_ref[0, pl.ds(p0, T), :]
    yt = y_ref[0]
    xt16 = xt.astype(_CT)

    Wx = w_ref[:, 0:H]
    Wy = w_ref[:, H:2 * H]
    Wc = w_ref[:, 2 * H:3 * H]

    dn = (((1,), (1,)), ((), ()))
    xp = jax.lax.dot_general(xt, Wx, dn, preferred_element_type=jnp.float32)
    yq = jax.lax.dot_general(yt, Wy, dn, preferred_element_type=jnp.float32)
    yq = yq + b_ref[...]

    def emit(chunk_ctx, nc=4):
        tc = T // nc
        yq3 = jax.lax.broadcast_in_dim(yq, (tc, T, H), (1, 2))
        Wc16 = Wc.astype(_CT)
        for i in range(nc):
            chunk = chunk_ctx(i, tc).reshape(tc * T, H)
            mm = jax.lax.dot_general(chunk, Wc16, dn,
                                     preferred_element_type=jnp.float32)
            mm = mm.reshape(tc, T, H)
            xp3 = jax.lax.broadcast_in_dim(xp[i * tc:(i + 1) * tc],
                                           (tc, T, H), (0, 2))
            o_ref[0, i * tc:(i + 1) * tc] = jnp.maximum(mm + xp3 + yq3, 0.0)

    def offdiag():
        xq = x_ref[0, pl.ds(q0, T), :]
        lo = jnp.minimum(p0, q0) + T
        hi = jnp.maximum(p0, q0)
        idx = jax.lax.broadcasted_iota(jnp.int32, (L, 1), 0)
        gap = jnp.max(jnp.where((idx >= lo) & (idx < hi), xrow, _NEG),
                      axis=0, keepdims=True)
        pre_p, suf_p, pre_q, suf_q = xt, xt, xq, xq
        s = 1
        while s < T:
            pad = jnp.full((s, H), _NEG, jnp.float32)
            pre_p = jnp.maximum(pre_p, jnp.concatenate([pad, pre_p[:-s]], 0))
            suf_p = jnp.maximum(suf_p, jnp.concatenate([suf_p[s:], pad], 0))
            pre_q = jnp.maximum(pre_q, jnp.concatenate([pad, pre_q[:-s]], 0))
            suf_q = jnp.maximum(suf_q, jnp.concatenate([suf_q[s:], pad], 0))
            s *= 2
        lt = p0 < q0
        A = jnp.where(lt, jnp.maximum(suf_p, gap), pre_p).astype(_CT)
        Bc = jnp.where(lt, pre_q, jnp.maximum(suf_q, gap)).astype(_CT)

        def chunk_ctx(i, tc):
            a3 = jax.lax.broadcast_in_dim(A[i * tc:(i + 1) * tc],
                                          (tc, T, H), (0, 2))
            b3 = jax.lax.broadcast_in_dim(Bc, (tc, T, H), (1, 2))
            return jnp.maximum(a3, b3)

        emit(chunk_ctx)

    def all_pairs(xb, Tb):
        if Tb <= 8:
            iloc = jax.lax.broadcasted_iota(jnp.int32, (Tb, 1), 0)
            levels = []
            k = 0
            while (1 << k) < Tb:
                half = 1 << k
                P = xb
                S = xb
                for jj in range(k):
                    sft = 1 << jj
                    canP = (iloc % half) >= sft
                    canS = (iloc % half) < (half - sft)
                    Psh = jnp.concatenate(
                        [jnp.full((sft, H), _NEG, jnp.float32), P[:-sft]], 0)
                    Ssh = jnp.concatenate(
                        [S[sft:], jnp.full((sft, H), _NEG, jnp.float32)], 0)
                    P = jnp.where(canP, jnp.maximum(P, Psh), P)
                    S = jnp.where(canS, jnp.maximum(S, Ssh), S)
                levels.append((S, P))
                k += 1
            pi = jax.lax.broadcasted_iota(jnp.int32, (Tb, Tb, H), 0)
            qi = jax.lax.broadcasted_iota(jnp.int32, (Tb, Tb, H), 1)
            v = pi ^ qi
            ltm = pi < qi
            xb3 = jax.lax.broadcast_in_dim(xb, (Tb, Tb, H), (0, 2))
            ctx = jnp.where(pi == qi, xb3, _NEG)
            for k, (S, P) in enumerate(levels):
                m3 = (v >> k) == 1
                Sr = jax.lax.broadcast_in_dim(S, (Tb, Tb, H), (0, 2))
                Sc = jax.lax.broadcast_in_dim(S, (Tb, Tb, H), (1, 2))
                Pr = jax.lax.broadcast_in_dim(P, (Tb, Tb, H), (0, 2))
                Pc = jax.lax.broadcast_in_dim(P, (Tb, Tb, H), (1, 2))
                upper = jnp.maximum(Sr, Pc)
                lower = jnp.maximum(Pr, Sc)
                ctx = jnp.where(m3, jnp.where(ltm, upper, lower), ctx)
            return ctx
        h = Tb // 2
        a = xb[:h]
        b = xb[h:]
        d0 = all_pairs(a, h)
        d1 = all_pairs(b, h)
        suf, pre = a, b
        s = 1
        while s < h:
            pad = jnp.full((s, H), _NEG, _CT)
            suf = jnp.maximum(suf, jnp.concatenate([suf[s:], pad], 0))
            pre = jnp.maximum(pre, jnp.concatenate([pad, pre[:-s]], 0))
            s *= 2
        sr = jax.lax.broadcast_in_dim(suf, (h, h, H), (0, 2))
        pc = jax.lax.broadcast_in_dim(pre, (h, h, H), (1, 2))
        pr = jax.lax.broadcast_in_dim(pre, (h, h, H), (0, 2))
        sc = jax.lax.broadcast_in_dim(suf, (h, h, H), (1, 2))
        cross_u = jnp.maximum(sr, pc)
        cross_l = jnp.maximum(pr, sc)
        top = jnp.concatenate([d0, cross_u], axis=1)
        bot = jnp.concatenate([cross_l, d1], axis=1)
        return jnp.concatenate([top, bot], axis=0)

    def build(off, Tb):
        if Tb <= 8:
            leaf = all_pairs(xt[off:off + Tb], Tb)
            ctx_sref[off:off + Tb, off:off + Tb, :] = leaf.astype(_CT)
            return
        xb = xt16[off:off + Tb]
        h = Tb // 2
        build(off, h)
        build(off + h, h)
        a = xb[:h]
        b = xb[h:]
        suf, pre = a, b
        s = 1
        while s < h:
            pad = jnp.full((s, H), _NEG, _CT)
            suf = jnp.maximum(suf, jnp.concatenate([suf[s:], pad], 0))
            pre = jnp.maximum(pre, jnp.concatenate([pad, pre[:-s]], 0))
            s *= 2
        sr = jax.lax.broadcast_in_dim(suf, (h, h, H), (0, 2))
        pc = jax.lax.broadcast_in_dim(pre, (h, h, H), (1, 2))
        pr = jax.lax.broadcast_in_dim(pre, (h, h, H), (0, 2))
        sc = jax.lax.broadcast_in_dim(suf, (h, h, H), (1, 2))
        ctx_sref[off:off + h, off + h:off + Tb, :] = jnp.maximum(sr, pc)
        ctx_sref[off + h:off + Tb, off:off + h, :] = jnp.maximum(pr, sc)

    def diag():
        build(0, T)
        emit(lambda i, tc: ctx_sref[i * tc:(i + 1) * tc])

    pl.when(tp == tq)(diag)
    pl.when(tp != tq)(offdiag)


def kernel(x, y, W_w, W_b):
    B, L, H = x.shape
    T = _T
    nT = L // T
    grid = (B, nT, nT)
    fn = functools.partial(_ctx_kernel, T=T, L=L, H=H)
    out = pl.pallas_call(
        fn,
        grid=grid,
        in_specs=[
            pl.BlockSpec((1, L, H), lambda b, i, j: (b, 0, 0)),
            pl.BlockSpec((1, T, H), lambda b, i, j: (b, j, 0)),
            pl.BlockSpec((H, 3 * H), lambda b, i, j: (0, 0)),
            pl.BlockSpec((1, H), lambda b, i, j: (0, 0)),
        ],
        out_specs=pl.BlockSpec((1, T, T, H), lambda b, i, j: (b, i, j, 0)),
        out_shape=jax.ShapeDtypeStruct((B, L, L, H), jnp.float32),
        scratch_shapes=[pltpu.VMEM((T, T, H), _CT)],
        compiler_params=pltpu.CompilerParams(
            dimension_semantics=("parallel", "parallel", "parallel")),
    )(x, y, W_w, W_b.reshape(1, H))
    return out


# speedup vs baseline: 1.1597x; 1.1597x over previous
import functools

import jax
import jax.numpy as jnp
from jax.experimental import pallas as pl
from jax.experimental.pallas import tpu as pltpu

_T = 128

_NEG = -jnp.inf


def _ctx_kernel(x_ref, y_ref, w_ref, b_ref, o_ref, ctx_sref, *, T, L, H):
    tp = pl.program_id(1)
    tq = pl.program_id(2)
    p0 = tp * T
    q0 = tq * T

    xrow = x_ref[0]
    xt = x_ref[0, pl.ds(p0, T), :]
    yt = y_ref[0]

    Wx = w_ref[:, 0:H]
    Wy = w_ref[:, H:2 * H]
    Wc = w_ref[:, 2 * H:3 * H]

    dn = (((1,), (1,)), ((), ()))
    xp = jax.lax.dot_general(xt, Wx, dn, preferred_element_type=jnp.float32)
    yq = jax.lax.dot_general(yt, Wy, dn, preferred_element_type=jnp.float32)
    yq = yq + b_ref[...]

    def emit(chunk_ctx, nc=4):
        tc = T // nc
        yq3 = jax.lax.broadcast_in_dim(yq, (tc, T, H), (1, 2))
        for i in range(nc):
            chunk = chunk_ctx(i, tc).reshape(tc * T, H)
            mm = jax.lax.dot_general(chunk, Wc, dn,
                                     preferred_element_type=jnp.float32)
            mm = mm.reshape(tc, T, H)
            xp3 = jax.lax.broadcast_in_dim(xp[i * tc:(i + 1) * tc],
                                           (tc, T, H), (0, 2))
            o_ref[0, i * tc:(i + 1) * tc] = jnp.maximum(mm + xp3 + yq3, 0.0)

    def offdiag():
        xq = x_ref[0, pl.ds(q0, T), :]
        lo = jnp.minimum(p0, q0) + T
        hi = jnp.maximum(p0, q0)
        idx = jax.lax.broadcasted_iota(jnp.int32, (L, 1), 0)
        gap = jnp.max(jnp.where((idx >= lo) & (idx < hi), xrow, _NEG),
                      axis=0, keepdims=True)
        pre_p, suf_p, pre_q, suf_q = xt, xt, xq, xq
        s = 1
        while s < T:
            pad = jnp.full((s, H), _NEG, jnp.float32)
            pre_p = jnp.maximum(pre_p, jnp.concatenate([pad, pre_p[:-s]], 0))
            suf_p = jnp.maximum(suf_p, jnp.concatenate([suf_p[s:], pad], 0))
            pre_q = jnp.maximum(pre_q, jnp.concatenate([pad, pre_q[:-s]], 0))
            suf_q = jnp.maximum(suf_q, jnp.concatenate([suf_q[s:], pad], 0))
            s *= 2
        lt = p0 < q0
        A = jnp.where(lt, jnp.maximum(suf_p, gap), pre_p)
        Bc = jnp.where(lt, pre_q, jnp.maximum(suf_q, gap))

        def chunk_ctx(i, tc):
            a3 = jax.lax.broadcast_in_dim(A[i * tc:(i + 1) * tc],
                                          (tc, T, H), (0, 2))
            b3 = jax.lax.broadcast_in_dim(Bc, (tc, T, H), (1, 2))
            return jnp.maximum(a3, b3)

        emit(chunk_ctx)

    def all_pairs(xb, Tb):
        if Tb <= 8:
            iloc = jax.lax.broadcasted_iota(jnp.int32, (Tb, 1), 0)
            levels = []
            k = 0
            while (1 << k) < Tb:
                half = 1 << k
                P = xb
                S = xb
                for jj in range(k):
                    sft = 1 << jj
                    canP = (iloc % half) >= sft
                    canS = (iloc % half) < (half - sft)
                    Psh = jnp.concatenate(
                        [jnp.full((sft, H), _NEG, jnp.float32), P[:-sft]], 0)
                    Ssh = jnp.concatenate(
                        [S[sft:], jnp.full((sft, H), _NEG, jnp.float32)], 0)
                    P = jnp.where(canP, jnp.maximum(P, Psh), P)
                    S = jnp.where(canS, jnp.maximum(S, Ssh), S)
                levels.append((S, P))
                k += 1
            pi = jax.lax.broadcasted_iota(jnp.int32, (Tb, Tb, H), 0)
            qi = jax.lax.broadcasted_iota(jnp.int32, (Tb, Tb, H), 1)
            v = pi ^ qi
            ltm = pi < qi
            xb3 = jax.lax.broadcast_in_dim(xb, (Tb, Tb, H), (0, 2))
            ctx = jnp.where(pi == qi, xb3, _NEG)
            for k, (S, P) in enumerate(levels):
                m3 = (v >> k) == 1
                Sr = jax.lax.broadcast_in_dim(S, (Tb, Tb, H), (0, 2))
                Sc = jax.lax.broadcast_in_dim(S, (Tb, Tb, H), (1, 2))
                Pr = jax.lax.broadcast_in_dim(P, (Tb, Tb, H), (0, 2))
                Pc = jax.lax.broadcast_in_dim(P, (Tb, Tb, H), (1, 2))
                upper = jnp.maximum(Sr, Pc)
                lower = jnp.maximum(Pr, Sc)
                ctx = jnp.where(m3, jnp.where(ltm, upper, lower), ctx)
            return ctx
        h = Tb // 2
        a = xb[:h]
        b = xb[h:]
        d0 = all_pairs(a, h)
        d1 = all_pairs(b, h)
        suf, pre = a, b
        s = 1
        while s < h:
            pad = jnp.full((s, H), _NEG, jnp.float32)
            suf = jnp.maximum(suf, jnp.concatenate([suf[s:], pad], 0))
            pre = jnp.maximum(pre, jnp.concatenate([pad, pre[:-s]], 0))
            s *= 2
        sr = jax.lax.broadcast_in_dim(suf, (h, h, H), (0, 2))
        pc = jax.lax.broadcast_in_dim(pre, (h, h, H), (1, 2))
        pr = jax.lax.broadcast_in_dim(pre, (h, h, H), (0, 2))
        sc = jax.lax.broadcast_in_dim(suf, (h, h, H), (1, 2))
        cross_u = jnp.maximum(sr, pc)
        cross_l = jnp.maximum(pr, sc)
        top = jnp.concatenate([d0, cross_u], axis=1)
        bot = jnp.concatenate([cross_l, d1], axis=1)
        return jnp.concatenate([top, bot], axis=0)

    def build(off, Tb):
        if Tb <= 8:
            ctx_sref[off:off + Tb, off:off + Tb, :] = all_pairs(
                xt[off:off + Tb], Tb)
            return
        xb = xt[off:off + Tb]
        h = Tb // 2
        build(off, h)
        build(off + h, h)
        a = xb[:h]
        b = xb[h:]
        suf, pre = a, b
        s = 1
        while s < h:
            pad = jnp.full((s, H), _NEG, jnp.float32)
            suf = jnp.maximum(suf, jnp.concatenate([suf[s:], pad], 0))
            pre = jnp.maximum(pre, jnp.concatenate([pad, pre[:-s]], 0))
            s *= 2
        sr = jax.lax.broadcast_in_dim(suf, (h, h, H), (0, 2))
        pc = jax.lax.broadcast_in_dim(pre, (h, h, H), (1, 2))
        pr = jax.lax.broadcast_in_dim(pre, (h, h, H), (0, 2))
        sc = jax.lax.broadcast_in_dim(suf, (h, h, H), (1, 2))
        ctx_sref[off:off + h, off + h:off + Tb, :] = jnp.maximum(sr, pc)
        ctx_sref[off + h:off + Tb, off:off + h, :] = jnp.maximum(pr, sc)

    def diag():
        build(0, T)
        emit(lambda i, tc: ctx_sref[i * tc:(i + 1) * tc])

    pl.when(tp == tq)(diag)
    pl.when(tp != tq)(offdiag)


def kernel(x, y, W_w, W_b):
    B, L, H = x.shape
    T = _T
    nT = L // T
    grid = (B, nT, nT)
    fn = functools.partial(_ctx_kernel, T=T, L=L, H=H)
    out = pl.pallas_call(
        fn,
        grid=grid,
        in_specs=[
            pl.BlockSpec((1, L, H), lambda b, i, j: (b, 0, 0)),
            pl.BlockSpec((1, T, H), lambda b, i, j: (b, j, 0)),
            pl.BlockSpec((H, 3 * H), lambda b, i, j: (0, 0)),
            pl.BlockSpec((1, H), lambda b, i, j: (0, 0)),
        ],
        out_specs=pl.BlockSpec((1, T, T, H), lambda b, i, j: (b, i, j, 0)),
        out_shape=jax.ShapeDtypeStruct((B, L, L, H), jnp.float32),
        scratch_shapes=[pltpu.VMEM((T, T, H), jnp.float32)],
        compiler_params=pltpu.CompilerParams(
            dimension_semantics=("parallel", "parallel", "parallel")),
    )(x, y, W_w, W_b.reshape(1, H))
    return out
